# baseline (speedup 1.0000x reference)
import functools

import jax
import jax.numpy as jnp
from jax import lax
from jax.experimental import pallas as pl
from jax.experimental.pallas import tpu as pltpu
from jax.experimental.pallas import tpu_sc as plsc

N_RAYS = 65536
N_BINS = 256
N_SAMPLES = 256
L = 16
NC, NSUB = 2, 16
NW = NC * NSUB
RAYS_PER_W = N_RAYS // NW
RB = 128
NBLK = RAYS_PER_W // RB
F32 = jnp.float32
I32 = jnp.int32


def _recip(x):
    seed = plsc.bitcast(jnp.full((L,), 0x7EF311C3, I32) - plsc.bitcast(x, I32),
                        F32)
    r = seed * (F32(2.0) - x * seed)
    r = r * (F32(2.0) - x * r)
    r = r * (F32(2.0) - x * r)
    return r


def _body(ro_h, rd_h, w_h, out_h,
          ro_v, rd_v,
          near_v, hs_v, w_v, out_v, c0_v, c1_v, h0_v, h1_v, t0_v, t1_v):
    wid = lax.axis_index("c") * NSUB + lax.axis_index("s")
    ones_i = jnp.full((L,), 1, I32)
    zeros_i = jnp.full((L,), 0, I32)
    iota_i = lax.iota(I32, L)
    iota16 = iota_i * 16
    iota16_f = iota16.astype(F32)
    iota3 = iota_i * 3
    fifteen = jnp.full((L,), 15, I32)

    c_v = (c0_v, c1_v)
    hist_v = (h0_v, h1_v)
    tmp_v = (t0_v, t1_v)

    def block(blk, _):
        rbase = wid * RAYS_PER_W + blk * RB
        pltpu.sync_copy(w_h.at[pl.ds(rbase * N_BINS, RB * N_BINS)], w_v)
        pltpu.sync_copy(ro_h.at[pl.ds(rbase * 3, RB * 3)], ro_v)
        pltpu.sync_copy(rd_h.at[pl.ds(rbase * 3, RB * 3)], rd_v)

        c0_v[pl.ds(0, L)] = jnp.full((L,), 0.0, F32)
        c1_v[pl.ds(0, L)] = jnp.full((L,), 0.0, F32)

        for g in range(RB // L):
            sl = pl.ds(g * L, L)
            lo = None
            hi = None
            for ax in range(3):
                idx3 = iota3 + (g * (3 * L) + ax)
                o = plsc.load_gather(ro_v, [idx3])
                d = plsc.load_gather(rd_v, [idx3]) + F32(1e-15)
                tmin = (F32(-2.0) - o) / d
                tmax = (F32(2.0) - o) / d
                a_lo = jnp.where(tmin < tmax, tmin, tmax)
                a_hi = jnp.where(tmin > tmax, tmin, tmax)
                lo = a_lo if lo is None else jnp.maximum(lo, a_lo)
                hi = a_hi if hi is None else jnp.minimum(hi, a_hi)
            bad = hi < lo
            nr = jnp.where(bad, F32(1e9), lo)
            fr = jnp.where(bad, F32(1e9), hi)
            nr = jnp.maximum(nr, F32(0.05))
            near_v[sl] = nr
            hs_v[sl] = (fr - nr) * F32(1.0 / 256.0)

        def ray(i, _):
            rr = (i * 2, i * 2 + 1)
            for p in range(2):
                for cc in range(17):
                    hist_v[p][pl.ds(cc * L, L)] = zeros_i

            for p in range(2):
                wbase = rr[p] * N_BINS
                widx = iota16 + wbase
                accs = []
                acc = None
                for j in range(L):
                    vj = plsc.load_gather(w_v, [widx + j]) + F32(1e-5)
                    acc = vj if acc is None else acc + vj
                    accs.append(acc)
                tot = accs[L - 1]
                cum_t = plsc.cumsum(tot)
                offs = cum_t - tot
                tmp_v[p][pl.ds(0, L)] = cum_t
                s_spl = plsc.load_gather(tmp_v[p], [fifteen])
                inv_s = _recip(s_spl)
                for j in range(L):
                    cs = (accs[j] + offs) * inv_s
                    plsc.store_scatter(c_v[p], [iota16 + (j + 1)], cs)
                    m = cs * F32(256.0) - F32(0.5)
                    ti = m.astype(I32)
                    cl = ti + jnp.where(m > ti.astype(F32), 1, 0)
                    sidx = jnp.minimum(jnp.maximum(cl, 0), 256)
                    plsc.addupdate_scatter(hist_v[p], [sidx], ones_i)

            for p in range(2):
                rfull = jnp.full((L,), rr[p], I32)
                near_s = plsc.load_gather(near_v, [rfull])
                hs_s = plsc.load_gather(hs_v, [rfull])
                haccs = []
                acc = None
                for j in range(L):
                    hj = plsc.load_gather(hist_v[p], [iota16 + j])
                    acc = hj if acc is None else acc + hj
                    haccs.append(acc)
                tot = haccs[L - 1]
                hoffs = plsc.cumsum(tot) - tot
                obase = rr[p] * N_SAMPLES
                for j in range(L):
                    b = haccs[j] + hoffs
                    cgb = plsc.load_gather(c_v[p], [b])
                    ia = jnp.minimum(b + 1, 256)
                    cga = plsc.load_gather(c_v[p], [ia])
                    denom = cga - cgb
                    denom = jnp.where(denom < F32(1e-5), F32(1.0), denom)
                    u = (iota16_f + F32(j + 0.5)) * F32(1.0 / 256.0)
                    t = (u - cgb) * _recip(denom)
                    y = b.astype(F32) + t * (ia - b).astype(F32)
                    plsc.store_scatter(
                        out_v, [iota16 + (obase + j)], near_s + hs_s * y)
            return 0

        lax.fori_loop(0, RB // 2, ray, 0)
        pltpu.sync_copy(out_v, out_h.at[pl.ds(rbase * N_SAMPLES, RB * N_SAMPLES)])
        return 0

    lax.fori_loop(0, NBLK, block, 0)


@jax.jit
def kernel(rays_o, rays_d, weights):
    mesh = plsc.VectorSubcoreMesh(core_axis_name="c", subcore_axis_name="s")
    k = functools.partial(
        pl.kernel,
        out_type=jax.ShapeDtypeStruct((N_RAYS * N_SAMPLES,), F32),
        mesh=mesh,
        compiler_params=pltpu.CompilerParams(needs_layout_passes=False),
        scratch_types=[
            pltpu.VMEM((RB * 3,), F32),
            pltpu.VMEM((RB * 3,), F32),
            pltpu.VMEM((RB,), F32),
            pltpu.VMEM((RB,), F32),
            pltpu.VMEM((RB * N_BINS,), F32),
            pltpu.VMEM((RB * N_SAMPLES,), F32),
            pltpu.VMEM((272,), F32),
            pltpu.VMEM((272,), F32),
            pltpu.VMEM((272,), I32),
            pltpu.VMEM((272,), I32),
            pltpu.VMEM((L,), F32),
            pltpu.VMEM((L,), F32),
        ],
    )(_body)
    out = k(
        rays_o.astype(F32).reshape(N_RAYS * 3),
        rays_d.astype(F32).reshape(N_RAYS * 3),
        weights.astype(F32).reshape(N_RAYS * N_BINS),
    )
    return out.reshape(N_RAYS, N_SAMPLES)


# speedup vs baseline: 1.1746x; 1.1746x over previous
import functools

import jax
import jax.numpy as jnp
from jax import lax
from jax.experimental import pallas as pl
from jax.experimental.pallas import tpu as pltpu
from jax.experimental.pallas import tpu_sc as plsc

N_RAYS = 65536
N_BINS = 256
N_SAMPLES = 256
L = 16
NC, NSUB = 2, 16
NW = NC * NSUB
RAYS_PER_W = N_RAYS // NW
RB = 128
NBLK = RAYS_PER_W // RB
F32 = jnp.float32
I32 = jnp.int32


def _body(ro_h, rd_h, w_h, out_h,
          ro_v, rd_v,
          near_v, hs_v, w_v, out_v, c0_v, c1_v, h0_v, h1_v, t0_v, t1_v):
    wid = lax.axis_index("c") * NSUB + lax.axis_index("s")
    ones_i = jnp.full((L,), 1, I32)
    zeros_i = jnp.full((L,), 0, I32)
    iota_i = lax.iota(I32, L)
    iota16 = iota_i * 16
    iota16_f = iota16.astype(F32)
    iota3 = iota_i * 3
    fifteen = jnp.full((L,), 15, I32)

    c_v = (c0_v, c1_v)
    hist_v = (h0_v, h1_v)
    tmp_v = (t0_v, t1_v)

    def block(blk, _):
        rbase = wid * RAYS_PER_W + blk * RB
        pltpu.sync_copy(w_h.at[pl.ds(rbase * N_BINS, RB * N_BINS)], w_v)
        pltpu.sync_copy(ro_h.at[pl.ds(rbase * 3, RB * 3)], ro_v)
        pltpu.sync_copy(rd_h.at[pl.ds(rbase * 3, RB * 3)], rd_v)

        c0_v[pl.ds(0, L)] = jnp.full((L,), 0.0, F32)
        c1_v[pl.ds(0, L)] = jnp.full((L,), 0.0, F32)

        for g in range(RB // L):
            sl = pl.ds(g * L, L)
            lo = None
            hi = None
            for ax in range(3):
                idx3 = iota3 + (g * (3 * L) + ax)
                o = plsc.load_gather(ro_v, [idx3])
                d = plsc.load_gather(rd_v, [idx3]) + F32(1e-15)
                tmin = (F32(-2.0) - o) / d
                tmax = (F32(2.0) - o) / d
                a_lo = jnp.where(tmin < tmax, tmin, tmax)
                a_hi = jnp.where(tmin > tmax, tmin, tmax)
                lo = a_lo if lo is None else jnp.maximum(lo, a_lo)
                hi = a_hi if hi is None else jnp.minimum(hi, a_hi)
            bad = hi < lo
            nr = jnp.where(bad, F32(1e9), lo)
            fr = jnp.where(bad, F32(1e9), hi)
            nr = jnp.maximum(nr, F32(0.05))
            near_v[sl] = nr
            hs_v[sl] = (fr - nr) * F32(1.0 / 256.0)

        def ray(i, _):
            rr = (i * 2, i * 2 + 1)
            for p in range(2):
                for cc in range(16):
                    hist_v[p][pl.ds(cc * L, L)] = zeros_i

            for p in range(2):
                wbase = rr[p] * N_BINS
                widx = iota16 + wbase
                accs = []
                acc = None
                for j in range(L):
                    vj = plsc.load_gather(w_v, [widx + j])
                    acc = vj if acc is None else acc + vj
                    accs.append(acc)
                tot = accs[L - 1] + F32(16e-5)
                cum_t = plsc.cumsum(tot)
                offs = cum_t - tot
                tmp_v[p][pl.ds(0, L)] = cum_t
                s_spl = plsc.load_gather(tmp_v[p], [fifteen])
                inv_s = F32(1.0) / s_spl
                for j in range(L):
                    cs = (accs[j] + (offs + F32((j + 1) * 1e-5))) * inv_s
                    plsc.store_scatter(c_v[p], [iota16 + (j + 1)], cs)
                    m = cs * F32(256.0) - F32(0.5)
                    ti = m.astype(I32)
                    sidx = ti + jnp.where(m > ti.astype(F32), 1, 0)
                    plsc.addupdate_scatter(hist_v[p], [sidx], ones_i)

            for p in range(2):
                rfull = jnp.full((L,), rr[p], I32)
                near_s = plsc.load_gather(near_v, [rfull])
                hs_s = plsc.load_gather(hs_v, [rfull])
                haccs = []
                acc = None
                for j in range(L):
                    hj = plsc.load_gather(hist_v[p], [iota16 + j])
                    acc = hj if acc is None else acc + hj
                    haccs.append(acc)
                tot = haccs[L - 1]
                hoffs = plsc.cumsum(tot) - tot
                obase = rr[p] * N_SAMPLES
                for j in range(L):
                    b = haccs[j] + hoffs
                    cgb = plsc.load_gather(c_v[p], [b])
                    cga = plsc.load_gather(c_v[p], [b + 1])
                    denom = cga - cgb
                    denom = jnp.where(denom < F32(1e-5), F32(1.0), denom)
                    u = (iota16_f + F32(j + 0.5)) * F32(1.0 / 256.0)
                    t = (u - cgb) / denom
                    y = b.astype(F32) + t
                    plsc.store_scatter(
                        out_v, [iota16 + (obase + j)], near_s + hs_s * y)
            return 0

        lax.fori_loop(0, RB // 2, ray, 0)
        pltpu.sync_copy(out_v, out_h.at[pl.ds(rbase * N_SAMPLES, RB * N_SAMPLES)])
        return 0

    lax.fori_loop(0, NBLK, block, 0)


@jax.jit
def kernel(rays_o, rays_d, weights):
    mesh = plsc.VectorSubcoreMesh(core_axis_name="c", subcore_axis_name="s")
    k = functools.partial(
        pl.kernel,
        out_type=jax.ShapeDtypeStruct((N_RAYS * N_SAMPLES,), F32),
        mesh=mesh,
        compiler_params=pltpu.CompilerParams(needs_layout_passes=False),
        scratch_types=[
            pltpu.VMEM((RB * 3,), F32),
            pltpu.VMEM((RB * 3,), F32),
            pltpu.VMEM((RB,), F32),
            pltpu.VMEM((RB,), F32),
            pltpu.VMEM((RB * N_BINS,), F32),
            pltpu.VMEM((RB * N_SAMPLES,), F32),
            pltpu.VMEM((272,), F32),
            pltpu.VMEM((272,), F32),
            pltpu.VMEM((272,), I32),
            pltpu.VMEM((272,), I32),
            pltpu.VMEM((L,), F32),
            pltpu.VMEM((L,), F32),
        ],
    )(_body)
    out = k(
        rays_o.astype(F32).reshape(N_RAYS * 3),
        rays_d.astype(F32).reshape(N_RAYS * 3),
        weights.astype(F32).reshape(N_RAYS * N_BINS),
    )
    return out.reshape(N_RAYS, N_SAMPLES)


# speedup vs baseline: 1.5086x; 1.2843x over previous
import functools

import jax
import jax.numpy as jnp
from jax import lax
from jax.experimental import pallas as pl
from jax.experimental.pallas import tpu as pltpu
from jax.experimental.pallas import tpu_sc as plsc

N_RAYS = 65536
N_BINS = 256
N_SAMPLES = 256
L = 16
NC, NSUB = 2, 16
NW = NC * NSUB
RAYS_PER_W = N_RAYS // NW
RB = 128
NBLK = RAYS_PER_W // RB
F32 = jnp.float32
I32 = jnp.int32


def _body(ro_h, rd_h, w_h, out_h,
          ro_v, rd_v,
          near_v, hs_v, w_v, out_v, c0_v, c1_v, h0_v, h1_v, t0_v, t1_v):
    wid = lax.axis_index("c") * NSUB + lax.axis_index("s")
    ones_i = jnp.full((L,), 1, I32)
    zeros_i = jnp.full((L,), 0, I32)
    iota_i = lax.iota(I32, L)
    iota16 = iota_i * 16
    iota16_f = iota16.astype(F32)
    iota3 = iota_i * 3
    fifteen = jnp.full((L,), 15, I32)

    c_v = (c0_v, c1_v)
    hist_v = (h0_v, h1_v)
    tmp_v = (t0_v, t1_v)

    def block(blk, _):
        rbase = wid * RAYS_PER_W + blk * RB
        pltpu.sync_copy(w_h.at[pl.ds(rbase * N_BINS, RB * N_BINS)], w_v)
        pltpu.sync_copy(ro_h.at[pl.ds(rbase * 3, RB * 3)], ro_v)
        pltpu.sync_copy(rd_h.at[pl.ds(rbase * 3, RB * 3)], rd_v)

        c0_v[pl.ds(0, L)] = jnp.full((L,), 0.0, F32)
        c1_v[pl.ds(0, L)] = jnp.full((L,), 0.0, F32)

        for g in range(RB // L):
            sl = pl.ds(g * L, L)
            lo = None
            hi = None
            for ax in range(3):
                idx3 = iota3 + (g * (3 * L) + ax)
                o = plsc.load_gather(ro_v, [idx3])
                d = plsc.load_gather(rd_v, [idx3]) + F32(1e-15)
                tmin = (F32(-2.0) - o) / d
                tmax = (F32(2.0) - o) / d
                a_lo = jnp.where(tmin < tmax, tmin, tmax)
                a_hi = jnp.where(tmin > tmax, tmin, tmax)
                lo = a_lo if lo is None else jnp.maximum(lo, a_lo)
                hi = a_hi if hi is None else jnp.minimum(hi, a_hi)
            bad = hi < lo
            nr = jnp.where(bad, F32(1e9), lo)
            fr = jnp.where(bad, F32(1e9), hi)
            nr = jnp.maximum(nr, F32(0.05))
            near_v[sl] = nr
            hs_v[sl] = (fr - nr) * F32(1.0 / 256.0)

        def ray(i, _):
            rr = (i * 2, i * 2 + 1)
            for cc in range(16):
                for p in range(2):
                    hist_v[p][pl.ds(cc * L, L)] = zeros_i

            widx = [iota16 + rr[p] * N_BINS for p in range(2)]
            accs = ([], [])
            acc = [None, None]
            for j in range(L):
                for p in range(2):
                    vj = plsc.load_gather(w_v, [widx[p] + j])
                    acc[p] = vj if acc[p] is None else acc[p] + vj
                    accs[p].append(acc[p])
            tot = [accs[p][L - 1] + F32(16e-5) for p in range(2)]
            cum_t = [plsc.cumsum(tot[p]) for p in range(2)]
            offs = [cum_t[p] - tot[p] for p in range(2)]
            for p in range(2):
                tmp_v[p][pl.ds(0, L)] = cum_t[p]
            s_spl = [plsc.load_gather(tmp_v[p], [fifteen]) for p in range(2)]
            inv_s = [F32(1.0) / s_spl[p] for p in range(2)]
            for j in range(L):
                for p in range(2):
                    cs = (accs[p][j]
                          + (offs[p] + F32((j + 1) * 1e-5))) * inv_s[p]
                    plsc.store_scatter(c_v[p], [iota16 + (j + 1)], cs)
                    m = cs * F32(256.0) - F32(0.5)
                    ti = m.astype(I32)
                    sidx = ti + jnp.where(m > ti.astype(F32), 1, 0)
                    plsc.addupdate_scatter(hist_v[p], [sidx], ones_i)

            rfull = [jnp.full((L,), rr[p], I32) for p in range(2)]
            near_s = [plsc.load_gather(near_v, [rfull[p]]) for p in range(2)]
            hs_s = [plsc.load_gather(hs_v, [rfull[p]]) for p in range(2)]
            hacc = [None, None]
            bs = ([], [])
            hoffs = [None, None]
            for j in range(L):
                for p in range(2):
                    hj = plsc.load_gather(hist_v[p], [iota16 + j])
                    hacc[p] = hj if hacc[p] is None else hacc[p] + hj
                    bs[p].append(hacc[p])
            for p in range(2):
                hoffs[p] = plsc.cumsum(hacc[p]) - hacc[p]
            for j in range(L):
                for p in range(2):
                    bs[p][j] = bs[p][j] + hoffs[p]
            obase = [rr[p] * N_SAMPLES for p in range(2)]

            def issue(j, p):
                return (plsc.load_gather(c_v[p], [bs[p][j]]),
                        plsc.load_gather(c_v[p], [bs[p][j] + 1]))

            g = [issue(0, 0), issue(0, 1)]
            for j in range(L):
                ng = ([issue(j + 1, 0), issue(j + 1, 1)]
                      if j + 1 < L else None)
                u = (iota16_f + F32(j + 0.5)) * F32(1.0 / 256.0)
                for p in range(2):
                    cgb, cga = g[p]
                    denom = cga - cgb
                    denom = jnp.where(denom < F32(1e-5), F32(1.0), denom)
                    t = (u - cgb) / denom
                    y = bs[p][j].astype(F32) + t
                    plsc.store_scatter(
                        out_v, [iota16 + (obase[p] + j)],
                        near_s[p] + hs_s[p] * y)
                g = ng
            return 0

        lax.fori_loop(0, RB // 2, ray, 0)
        pltpu.sync_copy(out_v, out_h.at[pl.ds(rbase * N_SAMPLES, RB * N_SAMPLES)])
        return 0

    lax.fori_loop(0, NBLK, block, 0)


@jax.jit
def kernel(rays_o, rays_d, weights):
    mesh = plsc.VectorSubcoreMesh(core_axis_name="c", subcore_axis_name="s")
    k = functools.partial(
        pl.kernel,
        out_type=jax.ShapeDtypeStruct((N_RAYS * N_SAMPLES,), F32),
        mesh=mesh,
        compiler_params=pltpu.CompilerParams(needs_layout_passes=False),
        scratch_types=[
            pltpu.VMEM((RB * 3,), F32),
            pltpu.VMEM((RB * 3,), F32),
            pltpu.VMEM((RB,), F32),
            pltpu.VMEM((RB,), F32),
            pltpu.VMEM((RB * N_BINS,), F32),
            pltpu.VMEM((RB * N_SAMPLES,), F32),
            pltpu.VMEM((272,), F32),
            pltpu.VMEM((272,), F32),
            pltpu.VMEM((272,), I32),
            pltpu.VMEM((272,), I32),
            pltpu.VMEM((L,), F32),
            pltpu.VMEM((L,), F32),
        ],
    )(_body)
    out = k(
        rays_o.astype(F32).reshape(N_RAYS * 3),
        rays_d.astype(F32).reshape(N_RAYS * 3),
        weights.astype(F32).reshape(N_RAYS * N_BINS),
    )
    return out.reshape(N_RAYS, N_SAMPLES)


# speedup vs baseline: 1.5141x; 1.0037x over previous
import functools

import jax
import jax.numpy as jnp
from jax import lax
from jax.experimental import pallas as pl
from jax.experimental.pallas import tpu as pltpu
from jax.experimental.pallas import tpu_sc as plsc

N_RAYS = 65536
N_BINS = 256
N_SAMPLES = 256
L = 16
NC, NSUB = 2, 16
NW = NC * NSUB
RAYS_PER_W = N_RAYS // NW
RB = 64
NBLK = RAYS_PER_W // RB
F32 = jnp.float32
I32 = jnp.int32


def _body(ro_h, rd_h, w_h, out_h,
          ro_v, rd_v, near_v, hs_v,
          w_v0, w_v1, out_v0, out_v1,
          c0_v, c1_v, h0_v, h1_v, t0_v, t1_v,
          semw0, semw1, semo0, semo1):
    wid = lax.axis_index("c") * NSUB + lax.axis_index("s")
    ones_i = jnp.full((L,), 1, I32)
    zeros_i = jnp.full((L,), 0, I32)
    iota_i = lax.iota(I32, L)
    iota16 = iota_i * 16
    iota16_f = iota16.astype(F32)
    iota3 = iota_i * 3
    fifteen = jnp.full((L,), 15, I32)

    c_v = (c0_v, c1_v)
    hist_v = (h0_v, h1_v)
    tmp_v = (t0_v, t1_v)

    pltpu.sync_copy(ro_h.at[pl.ds(wid * (RAYS_PER_W * 3), RAYS_PER_W * 3)],
                    ro_v)
    pltpu.sync_copy(rd_h.at[pl.ds(wid * (RAYS_PER_W * 3), RAYS_PER_W * 3)],
                    rd_v)

    c0_v[pl.ds(0, L)] = jnp.full((L,), 0.0, F32)
    c1_v[pl.ds(0, L)] = jnp.full((L,), 0.0, F32)

    def nearfar(g, _):
        lo = None
        hi = None
        for ax in range(3):
            idx3 = iota3 + (g * (3 * L) + ax)
            o = plsc.load_gather(ro_v, [idx3])
            d = plsc.load_gather(rd_v, [idx3]) + F32(1e-15)
            tmin = (F32(-2.0) - o) / d
            tmax = (F32(2.0) - o) / d
            a_lo = jnp.where(tmin < tmax, tmin, tmax)
            a_hi = jnp.where(tmin > tmax, tmin, tmax)
            lo = a_lo if lo is None else jnp.maximum(lo, a_lo)
            hi = a_hi if hi is None else jnp.minimum(hi, a_hi)
        bad = hi < lo
        nr = jnp.where(bad, F32(1e9), lo)
        fr = jnp.where(bad, F32(1e9), hi)
        nr = jnp.maximum(nr, F32(0.05))
        near_v[pl.ds(g * L, L)] = nr
        hs_v[pl.ds(g * L, L)] = (fr - nr) * F32(1.0 / 256.0)
        return 0

    lax.fori_loop(0, RAYS_PER_W // L, nearfar, 0)

    def do_block(blk, w_v, out_v):
        gray = blk * RB

        def ray(i, _):
            rr = (i * 2, i * 2 + 1)
            for cc in range(16):
                for p in range(2):
                    hist_v[p][pl.ds(cc * L, L)] = zeros_i

            widx = [iota16 + rr[p] * N_BINS for p in range(2)]
            accs = ([], [])
            acc = [None, None]
            for j in range(L):
                for p in range(2):
                    vj = plsc.load_gather(w_v, [widx[p] + j])
                    acc[p] = vj if acc[p] is None else acc[p] + vj
                    accs[p].append(acc[p])
            tot = [accs[p][L - 1] + F32(16e-5) for p in range(2)]
            cum_t = [plsc.cumsum(tot[p]) for p in range(2)]
            offs = [cum_t[p] - tot[p] for p in range(2)]
            for p in range(2):
                tmp_v[p][pl.ds(0, L)] = cum_t[p]
            s_spl = [plsc.load_gather(tmp_v[p], [fifteen]) for p in range(2)]
            inv_s = [F32(1.0) / s_spl[p] for p in range(2)]
            for j in range(L):
                for p in range(2):
                    cs = (accs[p][j]
                          + (offs[p] + F32((j + 1) * 1e-5))) * inv_s[p]
                    plsc.store_scatter(c_v[p], [iota16 + (j + 1)], cs)
                    m = cs * F32(256.0) - F32(0.5)
                    ti = m.astype(I32)
                    sidx = ti + jnp.where(m > ti.astype(F32), 1, 0)
                    plsc.addupdate_scatter(hist_v[p], [sidx], ones_i)

            rfull = [gray + rr[p] + jnp.full((L,), 0, I32) for p in range(2)]
            near_s = [plsc.load_gather(near_v, [rfull[p]]) for p in range(2)]
            hs_s = [plsc.load_gather(hs_v, [rfull[p]]) for p in range(2)]
            hacc = [None, None]
            bs = ([], [])
            hoffs = [None, None]
            for j in range(L):
                for p in range(2):
                    hj = plsc.load_gather(hist_v[p], [iota16 + j])
                    hacc[p] = hj if hacc[p] is None else hacc[p] + hj
                    bs[p].append(hacc[p])
            for p in range(2):
                hoffs[p] = plsc.cumsum(hacc[p]) - hacc[p]
            for j in range(L):
                for p in range(2):
                    bs[p][j] = bs[p][j] + hoffs[p]
            obase = [rr[p] * N_SAMPLES for p in range(2)]

            def issue(j, p):
                return (plsc.load_gather(c_v[p], [bs[p][j]]),
                        plsc.load_gather(c_v[p], [bs[p][j] + 1]))

            g = [issue(0, 0), issue(0, 1)]
            for j in range(L):
                ng = ([issue(j + 1, 0), issue(j + 1, 1)]
                      if j + 1 < L else None)
                u = (iota16_f + F32(j + 0.5)) * F32(1.0 / 256.0)
                for p in range(2):
                    cgb, cga = g[p]
                    denom = cga - cgb
                    denom = jnp.where(denom < F32(1e-5), F32(1.0), denom)
                    t = (u - cgb) / denom
                    y = bs[p][j].astype(F32) + t
                    plsc.store_scatter(
                        out_v, [iota16 + (obase[p] + j)],
                        near_s[p] + hs_s[p] * y)
                g = ng
            return 0

        lax.fori_loop(0, RB // 2, ray, 0)

    wbase_h = wid * (RAYS_PER_W * N_BINS)
    obase_h = wid * (RAYS_PER_W * N_SAMPLES)

    def wcopy(blk, w_v, sem):
        return pltpu.async_copy(
            w_h.at[pl.ds(wbase_h + blk * (RB * N_BINS), RB * N_BINS)],
            w_v, sem)

    def ocopy(blk, out_v, sem):
        return pltpu.async_copy(
            out_v,
            out_h.at[pl.ds(obase_h + blk * (RB * N_SAMPLES),
                           RB * N_SAMPLES)],
            sem)

    def pair(t, _):
        b0 = t * 2
        b1 = b0 + 1
        h0 = wcopy(b0, w_v0, semw0)
        h1 = wcopy(b1, w_v1, semw1)
        h0.wait()
        do_block(b0, w_v0, out_v0)
        ho0 = ocopy(b0, out_v0, semo0)
        h1.wait()
        do_block(b1, w_v1, out_v1)
        ho1 = ocopy(b1, out_v1, semo1)
        ho0.wait()
        ho1.wait()
        return 0

    lax.fori_loop(0, NBLK // 2, pair, 0)


@jax.jit
def kernel(rays_o, rays_d, weights):
    mesh = plsc.VectorSubcoreMesh(core_axis_name="c", subcore_axis_name="s")
    k = functools.partial(
        pl.kernel,
        out_type=jax.ShapeDtypeStruct((N_RAYS * N_SAMPLES,), F32),
        mesh=mesh,
        compiler_params=pltpu.CompilerParams(needs_layout_passes=False),
        scratch_types=[
            pltpu.VMEM((RAYS_PER_W * 3,), F32),
            pltpu.VMEM((RAYS_PER_W * 3,), F32),
            pltpu.VMEM((RAYS_PER_W,), F32),
            pltpu.VMEM((RAYS_PER_W,), F32),
            pltpu.VMEM((RB * N_BINS,), F32),
            pltpu.VMEM((RB * N_BINS,), F32),
            pltpu.VMEM((RB * N_SAMPLES,), F32),
            pltpu.VMEM((RB * N_SAMPLES,), F32),
            pltpu.VMEM((272,), F32),
            pltpu.VMEM((272,), F32),
            pltpu.VMEM((272,), I32),
            pltpu.VMEM((272,), I32),
            pltpu.VMEM((L,), F32),
            pltpu.VMEM((L,), F32),
            pltpu.SemaphoreType.DMA,
            pltpu.SemaphoreType.DMA,
            pltpu.SemaphoreType.DMA,
            pltpu.SemaphoreType.DMA,
        ],
    )(_body)
    out = k(
        rays_o.astype(F32).reshape(N_RAYS * 3),
        rays_d.astype(F32).reshape(N_RAYS * 3),
        weights.astype(F32).reshape(N_RAYS * N_BINS),
    )
    return out.reshape(N_RAYS, N_SAMPLES)
